# Initial kernel scaffold; baseline (speedup 1.0000x reference)
import jax
import jax.numpy as jnp
from jax.experimental import pallas as pl


def kernel(x, edge_index, adj_values, W, b_lin, bias):
    raise NotImplementedError("write your pallas kernel here")


# speedup vs baseline: 3.0572x; 3.0572x over previous
import functools

import jax
import jax.numpy as jnp
from jax import lax
from jax.experimental import pallas as pl
from jax.experimental.pallas import tpu as pltpu
from jax.experimental.pallas import tpu_sc as plsc

NC = 2
NS = 16
NW = NC * NS
CH = 128
LN = 16


def _linear_body(x_ref, w_ref, b_ref, o_ref):
    o_ref[...] = lax.dot_general(
        x_ref[...], w_ref[...], (((1,), (1,)), ((), ())),
        preferred_element_type=jnp.float32) + b_ref[...]


def _final_body(p_ref, b_ref, o_ref):
    o_ref[...] = p_ref[0] + p_ref[1] + b_ref[...]


def _make_scatter(n_pad, d, ngroup, grp):
    rows_per_tile = n_pad // NS
    zr = min(64, rows_per_tile)
    mesh = plsc.VectorSubcoreMesh(
        core_axis_name="c", subcore_axis_name="s",
        num_cores=NC, num_subcores=NS)

    @functools.partial(
        pl.kernel,
        out_type=jax.ShapeDtypeStruct((NC, n_pad, d), jnp.float32),
        mesh=mesh,
        scratch_types=[
            pltpu.VMEM((grp, CH), jnp.int32),
            pltpu.VMEM((grp, CH), jnp.int32),
            pltpu.VMEM((grp, CH), jnp.float32),
            pltpu.VMEM((CH, d), jnp.float32),
            pltpu.VMEM((zr, d), jnp.float32),
            pltpu.VMEM_SHARED((n_pad, d), jnp.float32),
            pltpu.SemaphoreType.DMA,
        ],
    )
    def scatter(support_hbm, src_hbm, dst_hbm, adj_hbm, out_hbm,
                src_v, dst_v, adj_v, rows_v, zbuf, acc, sem):
        c = lax.axis_index("c")
        s = lax.axis_index("s")
        wid = c * NS + s
        row0 = s * rows_per_tile

        zero16 = jnp.zeros((LN,), jnp.float32)

        def _zrow(r, carry):
            for k in range(d // LN):
                zbuf[r, pl.ds(k * LN, LN)] = zero16
            return carry

        lax.fori_loop(0, zr, _zrow, 0)
        for i in range(rows_per_tile // zr):
            pltpu.sync_copy(zbuf, acc.at[pl.ds(row0 + i * zr, zr)])
        rem = rows_per_tile % zr
        if rem:
            pltpu.sync_copy(zbuf.at[pl.ds(0, rem)],
                            acc.at[pl.ds(row0 + (rows_per_tile // zr) * zr, rem)])

        plsc.subcore_barrier()

        def _group(g, carry):
            pltpu.sync_copy(src_hbm.at[wid, pl.ds(g * grp, grp)], src_v)
            pltpu.sync_copy(dst_hbm.at[wid, pl.ds(g * grp, grp)], dst_v)
            pltpu.sync_copy(adj_hbm.at[wid, pl.ds(g * grp, grp)], adj_v)

            def _chunk(j, ccarry):
                pltpu.async_copy(support_hbm.at[src_v.at[j]], rows_v, sem).wait()

                def _sixteen(g16, gcarry):
                    av = adj_v[j, pl.ds(g16 * LN, LN)]
                    for i in range(LN):
                        a = av[i]
                        for k in range(d // LN):
                            sl = pl.ds(k * LN, LN)
                            rows_v[g16 * LN + i, sl] = rows_v[g16 * LN + i, sl] * a
                    return gcarry

                lax.fori_loop(0, CH // LN, _sixteen, 0)
                pltpu.sync_copy(rows_v, acc.at[dst_v.at[j]], add=True)
                return ccarry

            lax.fori_loop(0, grp, _chunk, 0)
            return carry

        lax.fori_loop(0, ngroup, _group, 0)
        plsc.subcore_barrier()
        pltpu.sync_copy(acc.at[pl.ds(row0, rows_per_tile)],
                        out_hbm.at[c, pl.ds(row0, rows_per_tile)])

    return scatter


def kernel(x, edge_index, adj_values, W, b_lin, bias):
    n, d = x.shape
    e = adj_values.shape[0]

    bm = 1000
    assert n % bm == 0
    support = pl.pallas_call(
        _linear_body,
        grid=(n // bm,),
        in_specs=[
            pl.BlockSpec((bm, d), lambda i: (i, 0)),
            pl.BlockSpec((d, d), lambda i: (0, 0)),
            pl.BlockSpec((1, d), lambda i: (0, 0)),
        ],
        out_specs=pl.BlockSpec((bm, d), lambda i: (i, 0)),
        out_shape=jax.ShapeDtypeStruct((n, d), jnp.float32),
    )(x, W, b_lin[None, :].astype(jnp.float32))

    dst = edge_index[0].astype(jnp.int32)
    src = edge_index[1].astype(jnp.int32)
    adj = adj_values.astype(jnp.float32)

    grp = 8
    epg = NW * CH * grp
    e_pad = -(-e // epg) * epg
    ngroup = e_pad // epg
    nchunk = ngroup * grp
    pad = e_pad - e
    if pad:
        src = jnp.concatenate([src, jnp.zeros((pad,), jnp.int32)])
        dst = jnp.concatenate([dst, jnp.zeros((pad,), jnp.int32)])
        adj = jnp.concatenate([adj, jnp.zeros((pad,), jnp.float32)])
    src3 = src.reshape(NW, nchunk, CH)
    dst3 = dst.reshape(NW, nchunk, CH)
    adj3 = adj.reshape(NW, nchunk, CH)

    n_pad = -(-n // (NS * 8)) * (NS * 8)
    partials = _make_scatter(n_pad, d, ngroup, grp)(support, src3, dst3, adj3)

    out = pl.pallas_call(
        _final_body,
        grid=(n // bm,),
        in_specs=[
            pl.BlockSpec((NC, bm, d), lambda i: (0, i, 0)),
            pl.BlockSpec((1, d), lambda i: (0, 0)),
        ],
        out_specs=pl.BlockSpec((bm, d), lambda i: (i, 0)),
        out_shape=jax.ShapeDtypeStruct((n, d), jnp.float32),
    )(partials, bias[None, :].astype(jnp.float32))
    return out
